# baseline (speedup 1.0000x reference)
import functools

import jax
import jax.numpy as jnp
from jax import lax
from jax.experimental import pallas as pl
from jax.experimental.pallas import tpu as pltpu
from jax.experimental.pallas import tpu_sc as plsc

_EMB_SIZE = 1000000
_EMB_DIM = 64
_B = 16384
_CTX = 20
_NC = 2
_NS = 16
_NW = _NC * _NS
_BPW = _B // _NW
_E = 16
_CHUNKS = _BPW // _E

_LN2 = 0.6931471805599453
_SQRT2 = 1.4142135623730951

_DNUMS = lax.GatherDimensionNumbers(
    offset_dims=(), collapsed_slice_dims=(0,), start_index_map=(0,))


def _log_pos(a):
    i = lax.bitcast_convert_type(a, jnp.int32)
    e = lax.shift_right_arithmetic(i, 23) - 127
    m = lax.bitcast_convert_type(
        jnp.bitwise_or(jnp.bitwise_and(i, 0x007FFFFF), 0x3F800000), jnp.float32)
    big = m > _SQRT2
    m = jnp.where(big, m * 0.5, m)
    e = jnp.where(big, e + 1, e)
    t = (m - 1.0) / (m + 1.0)
    t2 = t * t
    series = 1.0 + t2 * (1.0 / 3.0 + t2 * (1.0 / 5.0 + t2 * (
        1.0 / 7.0 + t2 * (1.0 / 9.0 + t2 * (1.0 / 11.0)))))
    return e.astype(jnp.float32) * _LN2 + 2.0 * t * series


def _lane_sum_splat(v):
    idx = jnp.arange(16, dtype=jnp.int32)
    for s in (1, 2, 4, 8):
        perm = jnp.bitwise_xor(idx, s)
        v = v + lax.gather(v, perm[:, None], dimension_numbers=_DNUMS,
                           slice_sizes=(1,),
                           mode=lax.GatherScatterMode.PROMISE_IN_BOUNDS)
    return v


def _dyn_splat(v, esplat):
    return lax.gather(v, esplat[:, None], dimension_numbers=_DNUMS,
                      slice_sizes=(1,),
                      mode=lax.GatherScatterMode.PROMISE_IN_BOUNDS)


def _logsigmoid(x):
    return -_log_pos(1.0 + jnp.exp(-x))


def _sc_body(w_ctx, w_word, ctx_t, pw_idx, neg_t, out,
             idx_ctx_o, idx_neg_o, idx_pw_o,
             phys_c, phys_n, phys_p,
             slab_ctx, slab_neg, rows_pw, out_v, sem):
    wid = lax.axis_index("s") * _NC + lax.axis_index("c")

    pltpu.sync_copy(ctx_t.at[:, pl.ds(wid * _BPW, _BPW)], idx_ctx_o)
    pltpu.sync_copy(neg_t.at[:, pl.ds(wid * _BPW, _BPW)], idx_neg_o)
    pltpu.sync_copy(pw_idx.at[pl.ds(wid * _BPW, _BPW)],
                    idx_pw_o.at[pl.ds(0, _BPW)])

    def chunk_body(c, acc):
        e0 = c * _E
        for j in range(_CTX):
            phys_c[j, :] = lax.shift_right_logical(
                idx_ctx_o[j, pl.ds(e0, 16)], 1)
            phys_n[j, :] = lax.shift_right_logical(
                idx_neg_o[j, pl.ds(e0, 16)], 1)
        phys_p[...] = lax.shift_right_logical(idx_pw_o[pl.ds(e0, 16)], 1)
        copies = []
        for j in range(_CTX):
            copies.append(pltpu.async_copy(
                w_ctx.at[phys_c.at[j]], slab_ctx.at[j], sem))
            copies.append(pltpu.async_copy(
                w_word.at[phys_n.at[j]], slab_neg.at[j], sem))
        copies.append(pltpu.async_copy(w_word.at[phys_p], rows_pw, sem))
        parc = [jnp.bitwise_and(idx_ctx_o[j, pl.ds(e0, 16)], 1)
                for j in range(_CTX)]
        parn = [jnp.bitwise_and(idx_neg_o[j, pl.ds(e0, 16)], 1)
                for j in range(_CTX)]
        parp = jnp.bitwise_and(idx_pw_o[pl.ds(e0, 16)], 1)
        for cp in copies:
            cp.wait()

        def ex_body(e, acc2):
            esp = jnp.full((16,), e, jnp.int32)

            def pick(slab, j, p, dc):
                lo = slab[j, e, pl.ds(dc * 16, 16)]
                hi = slab[j, e, pl.ds(64 + dc * 16, 16)]
                return lo + p * (hi - lo)

            mc = _dyn_splat(parc[0], esp).astype(jnp.float32)
            mn = _dyn_splat(parn[0], esp).astype(jnp.float32)
            a = [pick(slab_ctx, 0, mc, dc) for dc in range(4)]
            nacc = [pick(slab_neg, 0, mn, dc) for dc in range(4)]
            for j in range(1, _CTX):
                mc = _dyn_splat(parc[j], esp).astype(jnp.float32)
                mn = _dyn_splat(parn[j], esp).astype(jnp.float32)
                for dc in range(4):
                    a[dc] = a[dc] + pick(slab_ctx, j, mc, dc)
                    nacc[dc] = nacc[dc] + pick(slab_neg, j, mn, dc)
            mp = _dyn_splat(parp, esp).astype(jnp.float32)

            def pickp(dc):
                lo = rows_pw[e, pl.ds(dc * 16, 16)]
                hi = rows_pw[e, pl.ds(64 + dc * 16, 16)]
                return lo + mp * (hi - lo)

            pvec = [pickp(dc) for dc in range(4)]
            sp = a[0] * pvec[0] + a[1] * pvec[1] + a[2] * pvec[2] + a[3] * pvec[3]
            sn = a[0] * nacc[0] + a[1] * nacc[1] + a[2] * nacc[2] + a[3] * nacc[3]
            spl_sp = _lane_sum_splat(sp)
            spl_sn = _lane_sum_splat(sn)
            x = jnp.where(jnp.arange(16, dtype=jnp.int32) < 8, spl_sp, -spl_sn)
            return acc2 + _logsigmoid(x)

        return lax.fori_loop(0, _E, ex_body, acc, unroll=False)

    accv = lax.fori_loop(0, _CHUNKS, chunk_body,
                         jnp.zeros((16,), jnp.float32), unroll=False)
    total = _lane_sum_splat(accv) * 0.125
    out_v[0, :] = jnp.where(jnp.arange(16, dtype=jnp.int32) == 0, total, 0.0)
    zeros = jnp.zeros((16,), jnp.float32)
    for r in range(1, 8):
        out_v[r, :] = zeros
    pltpu.sync_copy(out_v, out.at[wid])


@jax.jit
def _cbow_loss_sc(w_ctx, w_word, ctx_t, pw_idx, neg_t):
    mesh = plsc.VectorSubcoreMesh(core_axis_name="c", subcore_axis_name="s")
    kfn = functools.partial(
        pl.kernel, mesh=mesh,
        out_type=jax.ShapeDtypeStruct((_NW, 8, 16), jnp.float32),
        scratch_types=[
            pltpu.VMEM((_CTX, _BPW), jnp.int32),
            pltpu.VMEM((_CTX, _BPW), jnp.int32),
            pltpu.VMEM((_BPW + 16,), jnp.int32),
            pltpu.VMEM((_CTX, _E), jnp.int32),
            pltpu.VMEM((_CTX, _E), jnp.int32),
            pltpu.VMEM((_E,), jnp.int32),
            pltpu.VMEM((_CTX, _E, 2 * _EMB_DIM), jnp.float32),
            pltpu.VMEM((_CTX, _E, 2 * _EMB_DIM), jnp.float32),
            pltpu.VMEM((_E, 2 * _EMB_DIM), jnp.float32),
            pltpu.VMEM((8, 16), jnp.float32),
            pltpu.SemaphoreType.DMA,
        ],
    )(_sc_body)
    return kfn(w_ctx, w_word, ctx_t, pw_idx, neg_t)


def kernel(W_ctx, W_word, pos_context, pos_word, neg_word):
    w_ctx2 = W_ctx.reshape(_EMB_SIZE // 2, 2 * _EMB_DIM)
    w_word2 = W_word.reshape(_EMB_SIZE // 2, 2 * _EMB_DIM)
    partials = _cbow_loss_sc(w_ctx2, w_word2, pos_context.T, pos_word,
                             neg_word.T)
    return -jnp.sum(partials)
